# Initial kernel scaffold; baseline (speedup 1.0000x reference)
import jax
import jax.numpy as jnp
from jax.experimental import pallas as pl


def kernel(src_embedding, tgt_embedding, src, tgt):
    raise NotImplementedError("write your pallas kernel here")


# speedup vs baseline: 3.5757x; 3.5757x over previous
import functools

import jax
import jax.numpy as jnp
from jax.experimental import pallas as pl
from jax.experimental.pallas import tpu as pltpu

_HI = jax.lax.Precision.HIGHEST
_T = 128


def _dotg(a, b, dims, precision=_HI):
    return jax.lax.dot_general(a, b, (dims, ((), ())),
                               preferred_element_type=jnp.float32,
                               precision=precision)


def _tp(x):
    return jnp.transpose(x, (1, 0))


def _rank_to_slot(vec_ref, v_row, src_slot, n, slot):
    def body(t, _):
        i0 = t * _T
        vi = vec_ref[pl.ds(i0, _T), src_slot:src_slot + 1]
        jj = jax.lax.broadcasted_iota(jnp.int32, (_T, n), 1)
        ii = i0 + jax.lax.broadcasted_iota(jnp.int32, (_T, n), 0)
        beats = (v_row > vi) | ((v_row == vi) & (jj < ii))
        vec_ref[pl.ds(i0, _T), slot:slot + 1] = (
            jnp.sum(beats.astype(jnp.float32), axis=1, keepdims=True))
        return 0
    jax.lax.fori_loop(0, n // _T, body, 0, unroll=False)


def _onehot_tile(rank_row, i0, rows, n, k_true):
    rr = i0 + jax.lax.broadcasted_iota(jnp.int32, (rows, n), 0)
    oh = (rank_row == rr.astype(jnp.float32)) & (rr < k_true)
    return oh.astype(jnp.float32)


def _make_body(n, f, k1, k1p, k2, k2p):
    nt = n // _T

    def body(set_ref, tet_ref, spt_ref, tpt_ref, out_ref, corr_ref,
             sc_ref, vec_ref):
        tet0 = tet_ref[0]
        yy_row = _tp(jnp.sum(tet0 * tet0, axis=1, keepdims=True))

        def p1(t, carry):
            colsum, colmax = carry
            i0 = t * _T
            a = set_ref[0, pl.ds(i0, _T), :]
            inner = _dotg(a, tet0, (((1,), (1,))), precision=None)
            xxt = jnp.sum(a * a, axis=1, keepdims=True)
            st = (2.0 * inner - xxt) - yy_row
            sc_ref[pl.ds(i0, _T), :] = st
            m = jnp.max(st, axis=1, keepdims=True)
            e = jnp.exp(st - m)
            z = jnp.sum(e, axis=1, keepdims=True)
            colsum = colsum + jnp.sum(e / z, axis=0, keepdims=True)
            colmax = jnp.maximum(colmax, jnp.max(st, axis=0, keepdims=True))
            return colsum, colmax

        colsum_row, colmax = jax.lax.fori_loop(
            0, nt, p1,
            (jnp.zeros((1, n), jnp.float32), jnp.full((1, n), -1e30, jnp.float32)),
            unroll=False)

        def p2(t, colz):
            st = sc_ref[pl.ds(t * _T, _T), :]
            return colz + jnp.sum(jnp.exp(st - colmax), axis=0, keepdims=True)

        colz = jax.lax.fori_loop(0, nt, p2, jnp.zeros((1, n), jnp.float32),
                                 unroll=False)

        def p3(t, _):
            st = sc_ref[pl.ds(t * _T, _T), :]
            rs = jnp.sum(jnp.exp(st - colmax) / colz, axis=1, keepdims=True)
            vec_ref[pl.ds(t * _T, _T), 0:1] = rs
            return 0

        jax.lax.fori_loop(0, nt, p3, 0, unroll=False)

        rowsum_row = _tp(vec_ref[0:n, 0:1])
        vec_ref[0:n, 6:7] = _tp(colsum_row)

        _rank_to_slot(vec_ref, colsum_row, 6, n, 2)
        rankc_row = _tp(vec_ref[0:n, 2:3])

        tpt = tpt_ref[0]

        def gather_t(t, _):
            i0 = t * _T
            oht = _onehot_tile(rankc_row, i0, _T, n, k1)
            sc_ref[pl.ds(i0, _T), f:2 * f] = _dotg(oht, tet0, (((1,), (0,))))
            vec_ref[pl.ds(i0, _T), 8:11] = _dotg(oht, tpt, (((1,), (0,))))
            return 0

        jax.lax.fori_loop(0, k1p // _T, gather_t, 0, unroll=False)

        _rank_to_slot(vec_ref, rowsum_row, 0, n, 2)
        rankr_row = _tp(vec_ref[0:n, 2:3])

        set_v = set_ref[0]
        spt = spt_ref[0]

        def gather_s(t, _):
            i0 = t * _T
            ohs = _onehot_tile(rankr_row, i0, _T, n, k1)
            sc_ref[pl.ds(i0, _T), 0:f] = _dotg(ohs, set_v, (((1,), (0,))))
            vec_ref[pl.ds(i0, _T), 12:15] = _dotg(ohs, spt, (((1,), (0,))))
            return 0

        jax.lax.fori_loop(0, k1p // _T, gather_s, 0, unroll=False)

        te_ot = sc_ref[0:k1p, f:2 * f]
        yy2_row = _tp(jnp.sum(te_ot * te_ot, axis=1, keepdims=True))
        tpo_col = vec_ref[0:k1p, 8:11]

        def s2(t, _):
            i0 = t * _T
            a = sc_ref[pl.ds(i0, _T), 0:f]
            inner = _dotg(a, te_ot, (((1,), (1,))), precision=None)
            xxt = jnp.sum(a * a, axis=1, keepdims=True)
            st = (2.0 * inner - xxt) - yy2_row
            jl = jax.lax.broadcasted_iota(jnp.int32, (_T, k1p), 1)
            st = jnp.where(jl < k1, st, jnp.float32(-1e30))
            m2 = jnp.max(st, axis=1, keepdims=True)
            p = jnp.exp(st - m2)
            zz = jnp.sum(p, axis=1, keepdims=True)
            pd = p / zz
            valt = jnp.max(pd, axis=1, keepdims=True)
            ismax = (pd == valt) & (jl < k1)
            jstar = jnp.min(jnp.where(ismax, jl, k1p), axis=1, keepdims=True)
            am = (jl == jstar).astype(jnp.float32)
            cand = _dotg(am, tpo_col, (((1,), (0,))))
            vec_ref[pl.ds(i0, _T), 3:6] = cand
            ivalid = (i0 + jax.lax.broadcasted_iota(jnp.int32, (_T, 1), 0)) < k1
            vec_ref[pl.ds(i0, _T), 1:2] = jnp.where(
                ivalid, valt, jnp.float32(-1e30))
            return 0

        jax.lax.fori_loop(0, k1p // _T, s2, 0, unroll=False)

        val_row = _tp(vec_ref[0:k1p, 1:2])
        _rank_to_slot(vec_ref, val_row, 1, k1p, 2)
        rankv_row = _tp(vec_ref[0:k1p, 2:3])
        oh2 = _onehot_tile(rankv_row, 0, k2p, k1p, k2)

        out_ref[0] = _dotg(oh2, vec_ref[0:k1p, 12:15], (((1,), (0,))))
        corr_ref[0] = _dotg(oh2, vec_ref[0:k1p, 3:6], (((1,), (0,))))

    return body


@functools.partial(jax.jit, static_argnums=())
def kernel(src_embedding, tgt_embedding, src, tgt):
    b, f, n = src_embedding.shape
    overlap2 = 0.75
    k1 = int(n * 0.84 * overlap2)
    k2 = int(k1 * 0.52 * overlap2)
    k1p = -(-k1 // _T) * _T
    k2p = -(-k2 // _T) * _T

    se_t = jnp.transpose(src_embedding, (0, 2, 1))
    te_t = jnp.transpose(tgt_embedding, (0, 2, 1))
    sp_t = jnp.transpose(src, (0, 2, 1))
    tp_t = jnp.transpose(tgt, (0, 2, 1))

    out, corr = pl.pallas_call(
        _make_body(n, f, k1, k1p, k2, k2p),
        grid=(b,),
        in_specs=[
            pl.BlockSpec((1, n, f), lambda i: (i, 0, 0)),
            pl.BlockSpec((1, n, f), lambda i: (i, 0, 0)),
            pl.BlockSpec((1, n, 3), lambda i: (i, 0, 0)),
            pl.BlockSpec((1, n, 3), lambda i: (i, 0, 0)),
        ],
        out_specs=[
            pl.BlockSpec((1, k2p, 3), lambda i: (i, 0, 0)),
            pl.BlockSpec((1, k2p, 3), lambda i: (i, 0, 0)),
        ],
        out_shape=[
            jax.ShapeDtypeStruct((b, k2p, 3), jnp.float32),
            jax.ShapeDtypeStruct((b, k2p, 3), jnp.float32),
        ],
        scratch_shapes=[
            pltpu.VMEM((n, n), jnp.float32),
            pltpu.VMEM((n, 128), jnp.float32),
        ],
    )(se_t, te_t, sp_t, tp_t)

    src_out = jnp.transpose(out[:, :k2, :], (0, 2, 1))
    src_corr = jnp.transpose(corr[:, :k2, :], (0, 2, 1))
    return src_out, src_corr
